# Initial kernel scaffold; baseline (speedup 1.0000x reference)
import jax
import jax.numpy as jnp
from jax.experimental import pallas as pl


def kernel(inputs, table):
    raise NotImplementedError("write your pallas kernel here")


# speedup vs baseline: 3.8141x; 3.8141x over previous
import jax
import jax.numpy as jnp
from jax import lax
from jax.experimental import pallas as pl
from jax.experimental.pallas import tpu as pltpu
from jax.experimental.pallas import tpu_sc as plsc

_MAXV = 1024
_D = 32
_B = 4096
_HW = 196
_IDXB = 3 * _HW
_NC = 2
_NS = 16
_NW = _NC * _NS
_BPW = _B // _NW
_CB = 8
_CHUNKI = _CB * _IDXB
_OUT = _D * _HW
_OUTB = _OUT + 16


def _sc_body(tbl_hbm, idx_hbm, out_hbm, tbl_v, idx_v, out_v, sem):
    wid = lax.axis_index("s") * _NC + lax.axis_index("c")
    pltpu.sync_copy(tbl_hbm, tbl_v)
    idx_v[pl.ds(_CHUNKI, 16)] = jnp.zeros((16,), jnp.int32)
    lane = lax.iota(jnp.int32, 16)

    def batch_body(t, carry):
        b = wid * _BPW + t

        @pl.when(t % _CB == 0)
        def _():
            off = pl.multiple_of(b * _IDXB, 8)
            pltpu.sync_copy(idx_hbm.at[pl.ds(off, _CHUNKI)],
                            idx_v.at[pl.ds(0, _CHUNKI)])

        @pl.when(t >= 2)
        def _():
            pltpu.make_async_copy(out_v.at[pl.ds(0, _OUT)],
                                  out_hbm.at[pl.ds(0, _OUT)], sem).wait()

        sbase = (t % 2) * _OUTB
        ibase = (t % _CB) * _IDXB

        def do_group(p0):
            r0 = plsc.load_gather(idx_v, [lane + (ibase + p0)]) * _D
            r1 = (plsc.load_gather(idx_v, [lane + (ibase + _HW + p0)])
                  + _MAXV) * _D
            r2 = (plsc.load_gather(idx_v, [lane + (ibase + 2 * _HW + p0)])
                  + 2 * _MAXV) * _D
            for d in range(_D):
                v = (plsc.load_gather(tbl_v, [r0 + d])
                     + plsc.load_gather(tbl_v, [r1 + d])
                     + plsc.load_gather(tbl_v, [r2 + d]))
                plsc.store_scatter(out_v, [lane + (sbase + d * _HW + p0)], v)

        do_group(192)

        def g_body(g, carry2):
            do_group(g * 16)
            return carry2

        lax.fori_loop(0, 12, g_body, 0)
        pltpu.async_copy(out_v.at[pl.ds(sbase, _OUT)],
                         out_hbm.at[pl.ds(pl.multiple_of(b * _OUT, 8), _OUT)],
                         sem)
        return carry

    lax.fori_loop(0, _BPW, batch_body, 0)
    for _ in range(2):
        pltpu.make_async_copy(out_v.at[pl.ds(0, _OUT)],
                              out_hbm.at[pl.ds(0, _OUT)], sem).wait()


def kernel(inputs, table):
    idx = inputs.reshape(_B * _IDXB)
    tbl = table.reshape(-1)
    mesh = plsc.VectorSubcoreMesh(
        core_axis_name="c", subcore_axis_name="s",
        num_cores=_NC, num_subcores=_NS)
    out = pl.kernel(
        _sc_body,
        out_type=jax.ShapeDtypeStruct((_B * _OUT,), jnp.float32),
        mesh=mesh,
        compiler_params=pltpu.CompilerParams(needs_layout_passes=False),
        scratch_types=[
            pltpu.VMEM((3 * _MAXV * _D,), jnp.float32),
            pltpu.VMEM((_CHUNKI + 16,), jnp.int32),
            pltpu.VMEM((2 * _OUTB,), jnp.float32),
            pltpu.SemaphoreType.DMA,
        ],
    )(tbl, idx)
    return out.reshape(_B, _D, 14, 14)
